# baseline (speedup 1.0000x reference)
import functools

import jax
import jax.numpy as jnp
from jax import lax
from jax.experimental import pallas as pl
from jax.experimental.pallas import tpu as pltpu
from jax.experimental.pallas import tpu_sc as plsc

N, DIM = 8, 384
N_IMG = N * DIM
NC, NS = 2, 16
NW = NC * NS
IMG_PER_W = N_IMG // NW

_MESH = plsc.VectorSubcoreMesh(
    core_axis_name="c", subcore_axis_name="s", num_cores=NC, num_subcores=NS
)


@functools.partial(
    pl.kernel,
    out_type=(
        jax.ShapeDtypeStruct((N_IMG, 3072), jnp.float32),
        jax.ShapeDtypeStruct((N_IMG, 1024), jnp.float32),
    ),
    mesh=_MESH,
    compiler_params=pltpu.CompilerParams(needs_layout_passes=False),
    scratch_types=[
        pltpu.VMEM((4096,), jnp.float32),
        pltpu.VMEM((3072,), jnp.float32),
        pltpu.VMEM((1024,), jnp.float32),
    ],
)
def _split_sc(x_hbm, z_hbm, c_hbm, xbuf, zbuf, cbuf):
    wid = lax.axis_index("s") * NC + lax.axis_index("c")
    ev2 = lax.iota(jnp.int32, 16) * 2

    def body(t, carry):
        img = wid * IMG_PER_W + t
        pltpu.sync_copy(x_hbm.at[img], xbuf)
        for p in range(32):
            src, dst = 128 * p, 96 * p
            for k in range(4):
                zbuf[pl.ds(dst + 16 * k, 16)] = xbuf[pl.ds(src + 16 * k, 16)]
            for h in range(2):
                idx = ev2 + (src + 64 + 32 * h)
                zbuf[pl.ds(dst + 64 + 16 * h, 16)] = plsc.load_gather(xbuf, [idx])
                cbuf[pl.ds(32 * p + 16 * h, 16)] = plsc.load_gather(xbuf, [idx + 1])
        pltpu.sync_copy(zbuf, z_hbm.at[img])
        pltpu.sync_copy(cbuf, c_hbm.at[img])
        return carry

    lax.fori_loop(0, IMG_PER_W, body, 0)


def kernel(x):
    xr = x.reshape(N_IMG, 4096)
    z2, c2 = _split_sc(xr)
    x_coarse = c2.reshape(N, DIM, 32, 32)
    z = z2.reshape(N, DIM, 3072)
    return (x_coarse, z)
